# baseline (speedup 1.0000x reference)
import functools

import jax
import jax.numpy as jnp
from jax import lax
from jax.experimental import pallas as pl
from jax.experimental.pallas import tpu as pltpu
from jax.experimental.pallas import tpu_sc as plsc

_B, _L, _D, _V = 4096, 200, 64, 1000000
_N = _B * _L
_NC, _NS, _LANES = 2, 16, 16
_NW = _NC * _NS
_TOK_PER_W = _N // _NW
_C = 512
_KR = _C // 128
_CHUNKS = _TOK_PER_W // _C


def _sc_body(seq_hbm, pos_ids_hbm, neg_ids_hbm, table_hbm,
             pos_out_hbm, neg_out_hbm,
             seq_v, pos_rows, neg_rows, pos_idx, neg_idx, pos_lg, neg_lg,
             gsem, ssem):
    wid = lax.axis_index("s") * _NC + lax.axis_index("c")

    def chunk_body(ci, carry):
        base = wid * _TOK_PER_W + ci * _C
        row = wid * (_TOK_PER_W // 128) + ci * _KR

        pltpu.sync_copy(pos_ids_hbm.at[pl.ds(row, _KR), :], pos_idx)
        pltpu.sync_copy(neg_ids_hbm.at[pl.ds(row, _KR), :], neg_idx)

        copies = [pltpu.async_copy(seq_hbm.at[pl.ds(base, _C), :], seq_v, ssem)]
        for j in range(_KR):
            copies.append(pltpu.async_copy(
                table_hbm.at[pos_idx.at[j]],
                pos_rows.at[pl.ds(j * 128, 128), :], gsem))
            copies.append(pltpu.async_copy(
                table_hbm.at[neg_idx.at[j]],
                neg_rows.at[pl.ds(j * 128, 128), :], gsem))
        for cp in copies:
            cp.wait()

        def group_body(g, c2):
            t0 = g * _LANES
            trow = t0 + lax.iota(jnp.int32, _LANES)
            accp = jnp.zeros((_LANES,), jnp.float32)
            accn = jnp.zeros((_LANES,), jnp.float32)
            for d in range(_D):
                dcol = jnp.full((_LANES,), d, jnp.int32)
                sv = plsc.load_gather(seq_v, [trow, dcol])
                pv = plsc.load_gather(pos_rows, [trow, dcol])
                nv = plsc.load_gather(neg_rows, [trow, dcol])
                accp = accp + sv * pv
                accn = accn + sv * nv
            pos_lg[pl.ds(t0, _LANES)] = accp
            neg_lg[pl.ds(t0, _LANES)] = accn
            return c2

        lax.fori_loop(0, _C // _LANES, group_body, 0)

        pltpu.sync_copy(pos_lg, pos_out_hbm.at[pl.ds(base, _C)])
        pltpu.sync_copy(neg_lg, neg_out_hbm.at[pl.ds(base, _C)])
        return carry

    lax.fori_loop(0, _CHUNKS, chunk_body, 0)


@jax.jit
def _sc_logits(seq2d, pos2d, neg2d, table):
    mesh = plsc.VectorSubcoreMesh(core_axis_name="c", subcore_axis_name="s")
    k = pl.kernel(
        _sc_body,
        mesh=mesh,
        compiler_params=pltpu.CompilerParams(
            needs_layout_passes=False,
            use_tc_tiling_on_sc=False,
        ),
        out_type=(
            jax.ShapeDtypeStruct((_N,), jnp.float32),
            jax.ShapeDtypeStruct((_N,), jnp.float32),
        ),
        scratch_types=[
            pltpu.VMEM((_C, _D), jnp.float32),
            pltpu.VMEM((_C, _D), jnp.float32),
            pltpu.VMEM((_C, _D), jnp.float32),
            pltpu.VMEM((_KR, 128), jnp.int32),
            pltpu.VMEM((_KR, 128), jnp.int32),
            pltpu.VMEM((_C,), jnp.float32),
            pltpu.VMEM((_C,), jnp.float32),
            pltpu.SemaphoreType.DMA,
            pltpu.SemaphoreType.DMA,
        ],
    )
    return k(seq2d, pos2d, neg2d, table)


_G = 16
_ROWS = _N // 128
_BLK = _ROWS // _G


def _tc_body(pl_ref, nl_ref, ids_ref, out_ref, acc_ref, nv_ref):
    i = pl.program_id(0)

    @pl.when(i == 0)
    def _init():
        acc_ref[0] = 0.0
        nv_ref[0] = 0.0

    x = pl_ref[...]
    y = nl_ref[...]
    m = (ids_ref[...] != 0).astype(jnp.float32)
    sp = jnp.log1p(jnp.exp(-jnp.abs(x))) + jnp.maximum(-x, 0.0)
    sn = jnp.log1p(jnp.exp(-jnp.abs(y))) + jnp.maximum(y, 0.0)
    acc_ref[0] += ((sp + sn) * m).sum()
    nv_ref[0] += m.sum()

    @pl.when(i == _G - 1)
    def _fin():
        out_ref[0, 0] = acc_ref[0] / (2.0 * nv_ref[0])


@jax.jit
def _tc_loss(pos_lg, neg_lg, pos2d):
    f = pl.pallas_call(
        _tc_body,
        grid=(_G,),
        in_specs=[
            pl.BlockSpec((_BLK, 128), lambda i: (i, 0)),
            pl.BlockSpec((_BLK, 128), lambda i: (i, 0)),
            pl.BlockSpec((_BLK, 128), lambda i: (i, 0)),
        ],
        out_specs=pl.BlockSpec(memory_space=pltpu.SMEM),
        out_shape=jax.ShapeDtypeStruct((1, 1), jnp.float32),
        scratch_shapes=[
            pltpu.SMEM((1,), jnp.float32),
            pltpu.SMEM((1,), jnp.float32),
        ],
    )
    out = f(pos_lg.reshape(_ROWS, 128), neg_lg.reshape(_ROWS, 128), pos2d)
    return out[0, 0]


def kernel(sequence_output, positive_target_ids, negative_target_ids, item_embedding_table):
    seq2d = sequence_output.reshape(_N, _D)
    pos2d = positive_target_ids.reshape(_ROWS, 128)
    neg2d = negative_target_ids.reshape(_ROWS, 128)
    pos_lg, neg_lg = _sc_logits(seq2d, pos2d, neg2d, item_embedding_table)
    return _tc_loss(pos_lg, neg_lg, pos2d)


# speedup vs baseline: 2.2011x; 2.2011x over previous
import jax
import jax.numpy as jnp
from jax import lax
from jax.experimental import pallas as pl
from jax.experimental.pallas import tpu as pltpu
from jax.experimental.pallas import tpu_sc as plsc

_B, _L, _D, _V = 4096, 200, 64, 1000000
_N = _B * _L
_NC, _NS = 2, 16
_NW = _NC * _NS
_TOK_PER_W = _N // _NW
_C = 512
_KR = _C // 128
_CHUNKS = _TOK_PER_W // _C
_ROWS = _N // 128


def _sc_body(pos_ids_hbm, neg_ids_hbm, table_hbm, out_hbm,
             pos_rows, neg_rows, pos_idx, neg_idx, gsem, wsem):
    wid = lax.axis_index("s") * _NC + lax.axis_index("c")

    def chunk_body(ci, carry):
        base = wid * _TOK_PER_W + ci * _C
        row = wid * (_TOK_PER_W // 128) + ci * _KR

        pltpu.sync_copy(pos_ids_hbm.at[pl.ds(row, _KR), :], pos_idx)
        pltpu.sync_copy(neg_ids_hbm.at[pl.ds(row, _KR), :], neg_idx)

        copies = []
        for j in range(_KR):
            copies.append(pltpu.async_copy(
                table_hbm.at[pos_idx.at[j]],
                pos_rows.at[pl.ds(j * 128, 128), :], gsem))
            copies.append(pltpu.async_copy(
                table_hbm.at[neg_idx.at[j]],
                neg_rows.at[pl.ds(j * 128, 128), :], gsem))
        for cp in copies:
            cp.wait()

        wp = pltpu.async_copy(
            pos_rows, out_hbm.at[pl.ds(base, _C), pl.ds(0, _D)], wsem)
        wn = pltpu.async_copy(
            neg_rows, out_hbm.at[pl.ds(base, _C), pl.ds(_D, _D)], wsem)
        wp.wait()
        wn.wait()
        return carry

    lax.fori_loop(0, _CHUNKS, chunk_body, 0)


@jax.jit
def _sc_gather(pos2d, neg2d, table):
    mesh = plsc.VectorSubcoreMesh(core_axis_name="c", subcore_axis_name="s")
    k = pl.kernel(
        _sc_body,
        mesh=mesh,
        compiler_params=pltpu.CompilerParams(
            needs_layout_passes=False,
            use_tc_tiling_on_sc=False,
        ),
        out_type=jax.ShapeDtypeStruct((_N, 2 * _D), jnp.float32),
        scratch_types=[
            pltpu.VMEM((_C, _D), jnp.float32),
            pltpu.VMEM((_C, _D), jnp.float32),
            pltpu.VMEM((_KR, 128), jnp.int32),
            pltpu.VMEM((_KR, 128), jnp.int32),
            pltpu.SemaphoreType.DMA,
            pltpu.SemaphoreType.DMA,
        ],
    )
    return k(pos2d, neg2d, table)


_G = 100
_BLK = _ROWS // _G


def _tc_body(seq_ref, comb_ref, ids_ref, out_ref, acc_ref, nv_ref):
    i = pl.program_id(0)

    @pl.when(i == 0)
    def _init():
        acc_ref[0] = 0.0
        nv_ref[0] = 0.0

    s = seq_ref[...]
    pos = comb_ref[..., 0:_D]
    neg = comb_ref[..., _D:2 * _D]
    dp = jnp.sum(s * pos, axis=-1)
    dn = jnp.sum(s * neg, axis=-1)
    m = (ids_ref[...] != 0).astype(jnp.float32)
    sp = jnp.log1p(jnp.exp(-jnp.abs(dp))) + jnp.maximum(-dp, 0.0)
    sn = jnp.log1p(jnp.exp(-jnp.abs(dn))) + jnp.maximum(dn, 0.0)
    acc_ref[0] += ((sp + sn) * m).sum()
    nv_ref[0] += m.sum()

    @pl.when(i == _G - 1)
    def _fin():
        out_ref[0, 0] = acc_ref[0] / (2.0 * nv_ref[0])


@jax.jit
def _tc_loss(seq3, comb3, pos2d):
    f = pl.pallas_call(
        _tc_body,
        grid=(_G,),
        in_specs=[
            pl.BlockSpec((_BLK, 128, _D), lambda i: (i, 0, 0)),
            pl.BlockSpec((_BLK, 128, 2 * _D), lambda i: (i, 0, 0)),
            pl.BlockSpec((_BLK, 128), lambda i: (i, 0)),
        ],
        out_specs=pl.BlockSpec(memory_space=pltpu.SMEM),
        out_shape=jax.ShapeDtypeStruct((1, 1), jnp.float32),
        scratch_shapes=[
            pltpu.SMEM((1,), jnp.float32),
            pltpu.SMEM((1,), jnp.float32),
        ],
    )
    return f(seq3, comb3, pos2d)[0, 0]


def kernel(sequence_output, positive_target_ids, negative_target_ids, item_embedding_table):
    seq3 = sequence_output.reshape(_ROWS, 128, _D)
    pos2d = positive_target_ids.reshape(_ROWS, 128)
    neg2d = negative_target_ids.reshape(_ROWS, 128)
    comb = _sc_gather(pos2d, neg2d, item_embedding_table)
    comb3 = comb.reshape(_ROWS, 128, 2 * _D)
    return _tc_loss(seq3, comb3, pos2d)


# speedup vs baseline: 3.4494x; 1.5672x over previous
import jax
import jax.numpy as jnp
from jax import lax
from jax.experimental import pallas as pl
from jax.experimental.pallas import tpu as pltpu
from jax.experimental.pallas import tpu_sc as plsc

_B, _L, _D, _V = 4096, 200, 64, 1000000
_N = _B * _L
_NC, _NS = 2, 16
_NW = _NC * _NS
_TOK_PER_W = _N // _NW
_C = 512
_KR = _C // 128
_CHUNKS = _TOK_PER_W // _C
_ROWS = _N // 128


def _sc_body(pos_ids_hbm, neg_ids_hbm, table_hbm, out_hbm,
             pos_rows, neg_rows, pos_idx, neg_idx, gsem, wsem):
    wid = lax.axis_index("s") * _NC + lax.axis_index("c")

    def chunk_body(ci, carry):
        base = wid * _TOK_PER_W + ci * _C
        row = wid * (_TOK_PER_W // 128) + ci * _KR

        pltpu.sync_copy(pos_ids_hbm.at[pl.ds(row, _KR), :], pos_idx)
        pltpu.sync_copy(neg_ids_hbm.at[pl.ds(row, _KR), :], neg_idx)

        copies = []
        for j in range(_KR):
            copies.append(pltpu.async_copy(
                table_hbm.at[pos_idx.at[j]],
                pos_rows.at[pl.ds(j * 128, 128), :], gsem))
            copies.append(pltpu.async_copy(
                table_hbm.at[neg_idx.at[j]],
                neg_rows.at[pl.ds(j * 128, 128), :], gsem))
        for cp in copies:
            cp.wait()

        wp = pltpu.async_copy(
            pos_rows, out_hbm.at[pl.ds(base, _C), pl.ds(0, _D)], wsem)
        wn = pltpu.async_copy(
            neg_rows, out_hbm.at[pl.ds(base, _C), pl.ds(_D, _D)], wsem)
        wp.wait()
        wn.wait()
        return carry

    lax.fori_loop(0, _CHUNKS, chunk_body, 0)


@jax.jit
def _sc_gather(pos2d, neg2d, table):
    mesh = plsc.VectorSubcoreMesh(core_axis_name="c", subcore_axis_name="s")
    k = pl.kernel(
        _sc_body,
        mesh=mesh,
        compiler_params=pltpu.CompilerParams(
            needs_layout_passes=False,
            use_tc_tiling_on_sc=False,
        ),
        out_type=jax.ShapeDtypeStruct((_N, 2 * _D), jnp.float32),
        scratch_types=[
            pltpu.VMEM((_C, _D), jnp.float32),
            pltpu.VMEM((_C, _D), jnp.float32),
            pltpu.VMEM((_KR, 128), jnp.int32),
            pltpu.VMEM((_KR, 128), jnp.int32),
            pltpu.SemaphoreType.DMA,
            pltpu.SemaphoreType.DMA,
        ],
    )
    return k(pos2d, neg2d, table)


_G = 25
_LB = _L // _G


def _tc_body(seq_ref, comb_ref, ids_ref, out_ref, acc_ref, nv_ref):
    i = pl.program_id(0)

    @pl.when(i == 0)
    def _init():
        acc_ref[0] = 0.0
        nv_ref[0] = 0.0

    s = seq_ref[...]
    ct = jnp.swapaxes(comb_ref[...], 1, 2)
    p = ct[:, 0:_D, :]
    n = ct[:, _D:2 * _D, :]
    dp = jnp.sum(s * p, axis=1)
    dn = jnp.sum(s * n, axis=1)
    m = (ids_ref[...] != 0).astype(jnp.float32)
    sp = jnp.log1p(jnp.exp(-jnp.abs(dp))) + jnp.maximum(-dp, 0.0)
    sn = jnp.log1p(jnp.exp(-jnp.abs(dn))) + jnp.maximum(dn, 0.0)
    acc_ref[0] += ((sp + sn) * m).sum()
    nv_ref[0] += m.sum()

    @pl.when(i == _G - 1)
    def _fin():
        out_ref[0, 0] = acc_ref[0] / (2.0 * nv_ref[0])


@jax.jit
def _tc_loss(seq_t, comb3, ids_lm):
    f = pl.pallas_call(
        _tc_body,
        grid=(_G,),
        in_specs=[
            pl.BlockSpec((_LB, _D, _B), lambda i: (i, 0, 0)),
            pl.BlockSpec((_LB, _B, 2 * _D), lambda i: (i, 0, 0)),
            pl.BlockSpec((_LB, _B), lambda i: (i, 0)),
        ],
        out_specs=pl.BlockSpec(memory_space=pltpu.SMEM),
        out_shape=jax.ShapeDtypeStruct((1, 1), jnp.float32),
        scratch_shapes=[
            pltpu.SMEM((1,), jnp.float32),
            pltpu.SMEM((1,), jnp.float32),
        ],
    )
    return f(seq_t, comb3, ids_lm)[0, 0]


def kernel(sequence_output, positive_target_ids, negative_target_ids, item_embedding_table):
    seq_t = jnp.transpose(sequence_output, (1, 2, 0))
    pos_lm = jnp.transpose(positive_target_ids, (1, 0))
    neg_lm = jnp.transpose(negative_target_ids, (1, 0))
    pos2d = pos_lm.reshape(_N // 128, 128)
    neg2d = neg_lm.reshape(_N // 128, 128)
    comb = _sc_gather(pos2d, neg2d, item_embedding_table)
    comb3 = comb.reshape(_L, _B, 2 * _D)
    return _tc_loss(seq_t, comb3, pos_lm)
